# baseline (speedup 1.0000x reference)
import functools

import jax
import jax.numpy as jnp
from jax import lax
from jax.experimental import pallas as pl
from jax.experimental.pallas import tpu as pltpu
from jax.experimental.pallas import tpu_sc as plsc

NUM_ROWS = 17
D = 128
CHUNK = 128
NBUF = 5
AHEAD = 3


@functools.lru_cache(maxsize=None)
def _build(batch: int):
    info = plsc.get_sparse_core_info()
    nw = info.num_cores * info.num_subcores
    assert batch % (nw * NBUF * CHUNK) == 0
    b_per_w = batch // nw
    n_chunks = b_per_w // CHUNK
    mesh = plsc.VectorSubcoreMesh(core_axis_name="c", subcore_axis_name="s")

    @functools.partial(
        pl.kernel,
        out_type=jax.ShapeDtypeStruct((batch, D), jnp.float32),
        mesh=mesh,
        scratch_types=[
            pltpu.VMEM_SHARED((NUM_ROWS, D), jnp.float32),
            pltpu.VMEM((n_chunks, CHUNK), jnp.int32),
        ]
        + [pltpu.VMEM((CHUNK, D), jnp.float32)] * NBUF
        + [pltpu.SemaphoreType.DMA] * (2 * NBUF),
    )
    def gather_kernel(idx_hbm, table_hbm, out_hbm, table_sh, idx_all, *bufs):
        rows = bufs[:NBUF]
        gsem = bufs[NBUF:2 * NBUF]
        osem = bufs[2 * NBUF:]
        wid = lax.axis_index("s") * info.num_cores + lax.axis_index("c")
        base = wid * b_per_w

        @pl.when(lax.axis_index("s") == 0)
        def _stage_table():
            pltpu.sync_copy(table_hbm, table_sh)

        pltpu.sync_copy(
            idx_hbm.at[pl.ds(wid * n_chunks, n_chunks)], idx_all)
        plsc.subcore_barrier()

        def start_gather(g, b):
            pltpu.async_copy(table_hbm.at[idx_all.at[g]], rows[b], gsem[b])

        def wait_gather(b):
            pltpu.make_async_copy(
                table_hbm.at[idx_all.at[0]], rows[b], gsem[b]).wait()

        def start_out(g, b):
            pltpu.async_copy(
                rows[b], out_hbm.at[pl.ds(base + g * CHUNK, CHUNK)], osem[b])

        def wait_out(b):
            pltpu.make_async_copy(
                rows[b], out_hbm.at[pl.ds(base, CHUNK)], osem[b]).wait()

        for g0 in range(AHEAD):
            start_gather(g0, g0)

        def ring_body(gg, carry):
            for p in range(NBUF):
                g = gg * NBUF + p
                b = p
                b2 = (p + AHEAD) % NBUF

                @pl.when(g >= NBUF - AHEAD)
                def _drain_out():
                    wait_out(b2)

                @pl.when(g + AHEAD < n_chunks)
                def _prefetch():
                    start_gather(g + AHEAD, b2)

                wait_gather(b)
                start_out(g, b)
            return carry

        lax.fori_loop(0, n_chunks // NBUF, ring_body, 0)
        for gtail in range(n_chunks - (NBUF - AHEAD), n_chunks):
            wait_out(gtail % NBUF)

    return gather_kernel


def kernel(hierarchy_labels, weight):
    b, l = hierarchy_labels.shape
    idx = hierarchy_labels.reshape(-1, CHUNK).astype(jnp.int32)
    out = _build(b * l)(idx, weight)
    return out.reshape(b, l, D)


# speedup vs baseline: 13.5156x; 13.5156x over previous
import functools

import jax
import jax.numpy as jnp
from jax import lax
from jax.experimental import pallas as pl
from jax.experimental.pallas import tpu as pltpu
from jax.experimental.pallas import tpu_sc as plsc

NUM_ROWS = 17
D = 128
CHUNK = 128
NBUF = 5
AHEAD = 3


@functools.lru_cache(maxsize=None)
def _build(batch: int):
    info = plsc.get_sparse_core_info()
    nw = info.num_cores * info.num_subcores
    assert batch % (nw * NBUF * CHUNK) == 0
    b_per_w = batch // nw
    n_chunks = b_per_w // CHUNK
    mesh = plsc.VectorSubcoreMesh(core_axis_name="c", subcore_axis_name="s")

    @functools.partial(
        pl.kernel,
        out_type=jax.ShapeDtypeStruct((batch, D), jnp.float32),
        mesh=mesh,
        scratch_types=[
            pltpu.VMEM_SHARED((info.num_subcores * NUM_ROWS, D), jnp.float32),
            pltpu.VMEM((n_chunks, CHUNK), jnp.int32),
        ]
        + [pltpu.VMEM((CHUNK, D), jnp.float32)] * NBUF
        + [pltpu.SemaphoreType.DMA] * (2 * NBUF),
    )
    def gather_kernel(idx_hbm, table_hbm, out_hbm, table_sh, idx_all, *bufs):
        rows = bufs[:NBUF]
        gsem = bufs[NBUF:2 * NBUF]
        osem = bufs[2 * NBUF:]
        wid = lax.axis_index("s") * info.num_cores + lax.axis_index("c")
        base = wid * b_per_w

        sid = lax.axis_index("s")
        pltpu.sync_copy(table_hbm,
                        table_sh.at[pl.ds(sid * NUM_ROWS, NUM_ROWS)])
        pltpu.sync_copy(
            idx_hbm.at[pl.ds(wid * n_chunks, n_chunks)], idx_all)

        row_off = jnp.full((16,), sid * NUM_ROWS, jnp.int32)

        def offset_body(c, carry):
            for j in range(CHUNK // 16):
                sl = pl.ds(j * 16, 16)
                idx_all[c, sl] = idx_all[c, sl] + row_off
            return carry

        lax.fori_loop(0, n_chunks, offset_body, 0)

        def start_gather(g, b):
            pltpu.async_copy(table_sh.at[idx_all.at[g]], rows[b], gsem[b])

        def wait_gather(b):
            pltpu.make_async_copy(
                table_sh.at[idx_all.at[0]], rows[b], gsem[b]).wait()

        def start_out(g, b):
            pltpu.async_copy(
                rows[b], out_hbm.at[pl.ds(base + g * CHUNK, CHUNK)], osem[b])

        def wait_out(b):
            pltpu.make_async_copy(
                rows[b], out_hbm.at[pl.ds(base, CHUNK)], osem[b]).wait()

        for g0 in range(AHEAD):
            start_gather(g0, g0)

        def ring_body(gg, carry):
            for p in range(NBUF):
                g = gg * NBUF + p
                b = p
                b2 = (p + AHEAD) % NBUF

                @pl.when(g >= NBUF - AHEAD)
                def _drain_out():
                    wait_out(b2)

                @pl.when(g + AHEAD < n_chunks)
                def _prefetch():
                    start_gather(g + AHEAD, b2)

                wait_gather(b)
                start_out(g, b)
            return carry

        lax.fori_loop(0, n_chunks // NBUF, ring_body, 0)
        for gtail in range(n_chunks - (NBUF - AHEAD), n_chunks):
            wait_out(gtail % NBUF)

    return gather_kernel


def kernel(hierarchy_labels, weight):
    b, l = hierarchy_labels.shape
    idx = hierarchy_labels.reshape(-1, CHUNK).astype(jnp.int32)
    out = _build(b * l)(idx, weight)
    return out.reshape(b, l, D)


# speedup vs baseline: 13.6278x; 1.0083x over previous
import functools

import jax
import jax.numpy as jnp
from jax import lax
from jax.experimental import pallas as pl
from jax.experimental.pallas import tpu as pltpu
from jax.experimental.pallas import tpu_sc as plsc

NUM_ROWS = 17
D = 128
CHUNK = 64
NBUF = 8
AHEAD = 3


@functools.lru_cache(maxsize=None)
def _build(batch: int):
    info = plsc.get_sparse_core_info()
    nw = info.num_cores * info.num_subcores
    assert batch % (nw * NBUF * CHUNK) == 0
    b_per_w = batch // nw
    n_chunks = b_per_w // CHUNK
    mesh = plsc.VectorSubcoreMesh(core_axis_name="c", subcore_axis_name="s")

    @functools.partial(
        pl.kernel,
        out_type=jax.ShapeDtypeStruct((batch, D), jnp.float32),
        mesh=mesh,
        scratch_types=[
            pltpu.VMEM_SHARED((NUM_ROWS, D), jnp.float32),
            pltpu.VMEM((n_chunks, CHUNK), jnp.int32),
        ]
        + [pltpu.VMEM((CHUNK, D), jnp.float32)] * NBUF
        + [pltpu.SemaphoreType.DMA] * (2 * NBUF),
    )
    def gather_kernel(idx_hbm, table_hbm, out_hbm, table_sh, idx_all, *bufs):
        rows = bufs[:NBUF]
        gsem = bufs[NBUF:2 * NBUF]
        osem = bufs[2 * NBUF:]
        wid = lax.axis_index("s") * info.num_cores + lax.axis_index("c")
        base = wid * b_per_w

        @pl.when(lax.axis_index("s") == 0)
        def _stage_table():
            pltpu.sync_copy(table_hbm, table_sh)

        pltpu.sync_copy(
            idx_hbm.at[pl.ds(wid * n_chunks, n_chunks)], idx_all)
        plsc.subcore_barrier()

        def start_gather(g, b):
            pltpu.async_copy(table_sh.at[idx_all.at[g]], rows[b], gsem[b])

        def wait_gather(b):
            pltpu.make_async_copy(
                table_sh.at[idx_all.at[0]], rows[b], gsem[b]).wait()

        def start_out(g, b):
            pltpu.async_copy(
                rows[b], out_hbm.at[pl.ds(base + g * CHUNK, CHUNK)], osem[b])

        def wait_out(b):
            pltpu.make_async_copy(
                rows[b], out_hbm.at[pl.ds(base, CHUNK)], osem[b]).wait()

        for g0 in range(AHEAD):
            start_gather(g0, g0)

        def ring_body(gg, carry):
            for p in range(NBUF):
                g = gg * NBUF + p
                b = p
                b2 = (p + AHEAD) % NBUF

                @pl.when(g >= NBUF - AHEAD)
                def _drain_out():
                    wait_out(b2)

                @pl.when(g + AHEAD < n_chunks)
                def _prefetch():
                    start_gather(g + AHEAD, b2)

                wait_gather(b)
                start_out(g, b)
            return carry

        lax.fori_loop(0, n_chunks // NBUF, ring_body, 0)
        for gtail in range(n_chunks - (NBUF - AHEAD), n_chunks):
            wait_out(gtail % NBUF)

    return gather_kernel


def kernel(hierarchy_labels, weight):
    b, l = hierarchy_labels.shape
    idx = hierarchy_labels.reshape(-1, CHUNK).astype(jnp.int32)
    out = _build(b * l)(idx, weight)
    return out.reshape(b, l, D)


# speedup vs baseline: 13.6402x; 1.0009x over previous
import functools

import jax
import jax.numpy as jnp
from jax import lax
from jax.experimental import pallas as pl
from jax.experimental.pallas import tpu as pltpu
from jax.experimental.pallas import tpu_sc as plsc

NUM_ROWS = 17
D = 128
CHUNK = 64
NBUF = 8
AHEAD = 4


@functools.lru_cache(maxsize=None)
def _build(batch: int):
    info = plsc.get_sparse_core_info()
    nw = info.num_cores * info.num_subcores
    assert batch % (nw * NBUF * CHUNK) == 0
    b_per_w = batch // nw
    n_chunks = b_per_w // CHUNK
    mesh = plsc.VectorSubcoreMesh(core_axis_name="c", subcore_axis_name="s")

    @functools.partial(
        pl.kernel,
        out_type=jax.ShapeDtypeStruct((batch, D), jnp.float32),
        mesh=mesh,
        scratch_types=[
            pltpu.VMEM_SHARED((NUM_ROWS, D), jnp.float32),
            pltpu.VMEM((n_chunks, CHUNK), jnp.int32),
        ]
        + [pltpu.VMEM((CHUNK, D), jnp.float32)] * NBUF
        + [pltpu.SemaphoreType.DMA] * (2 * NBUF),
    )
    def gather_kernel(idx_hbm, table_hbm, out_hbm, table_sh, idx_all, *bufs):
        rows = bufs[:NBUF]
        gsem = bufs[NBUF:2 * NBUF]
        osem = bufs[2 * NBUF:]
        wid = lax.axis_index("s") * info.num_cores + lax.axis_index("c")
        base = wid * b_per_w

        @pl.when(lax.axis_index("s") == 0)
        def _stage_table():
            pltpu.sync_copy(table_hbm, table_sh)

        pltpu.sync_copy(
            idx_hbm.at[pl.ds(wid * n_chunks, n_chunks)], idx_all)
        plsc.subcore_barrier()

        def start_gather(g, b):
            pltpu.async_copy(table_sh.at[idx_all.at[g]], rows[b], gsem[b])

        def wait_gather(b):
            pltpu.make_async_copy(
                table_sh.at[idx_all.at[0]], rows[b], gsem[b]).wait()

        def start_out(g, b):
            pltpu.async_copy(
                rows[b], out_hbm.at[pl.ds(base + g * CHUNK, CHUNK)], osem[b])

        def wait_out(b):
            pltpu.make_async_copy(
                rows[b], out_hbm.at[pl.ds(base, CHUNK)], osem[b]).wait()

        for g0 in range(AHEAD):
            start_gather(g0, g0)

        def ring_body(gg, carry):
            for p in range(NBUF):
                g = gg * NBUF + p
                b = p
                b2 = (p + AHEAD) % NBUF

                @pl.when(g >= NBUF - AHEAD)
                def _drain_out():
                    wait_out(b2)

                @pl.when(g + AHEAD < n_chunks)
                def _prefetch():
                    start_gather(g + AHEAD, b2)

                wait_gather(b)
                start_out(g, b)
            return carry

        lax.fori_loop(0, n_chunks // NBUF, ring_body, 0)
        for gtail in range(n_chunks - (NBUF - AHEAD), n_chunks):
            wait_out(gtail % NBUF)

    return gather_kernel


def kernel(hierarchy_labels, weight):
    b, l = hierarchy_labels.shape
    idx = hierarchy_labels.reshape(-1, CHUNK).astype(jnp.int32)
    out = _build(b * l)(idx, weight)
    return out.reshape(b, l, D)
